# Initial kernel scaffold; baseline (speedup 1.0000x reference)
import jax
import jax.numpy as jnp
from jax.experimental import pallas as pl


def kernel(x, edge_index, W0, b0, W1, b1, W2, b2):
    raise NotImplementedError("write your pallas kernel here")


# speedup vs baseline: 8.7566x; 8.7566x over previous
import functools
import jax
import jax.numpy as jnp
from jax import lax
from jax.experimental import pallas as pl
from jax.experimental.pallas import tpu as pltpu
from jax.experimental.pallas import tpu_sc as plsc

_N = 10000
_D = 128
_E = 320000
_NC = 2
_NS = 16
_NW = _NC * _NS
_EPW = _E // _NW
_CH = 80
_NFULL = _EPW // _CH
_RT = 624
_TAIL0 = _NS * _RT
_TAILN = _N - _TAIL0
_ZR = 104

_f32 = jnp.float32


def _zero_fill(buf, rows, cols):
    @pl.loop(0, rows)
    def _(r):
        @pl.loop(0, cols, step=16)
        def _(c):
            buf[r, pl.ds(c, 16)] = jnp.zeros((16,), _f32)


def _agg_body(edge_hbm, g_hbm, out_hbm, sidx, didx, stage, zbuf, acc):
    cid = lax.axis_index("c")
    sid = lax.axis_index("s")
    wid = cid * _NS + sid
    ebase = wid * _EPW
    row0 = sid * _RT

    _zero_fill(zbuf, _ZR, _D)
    for j in range(6):
        pltpu.sync_copy(zbuf, acc.at[pl.ds(row0 + j * _ZR, _ZR)])

    @pl.when(sid == 0)
    def _():
        pltpu.sync_copy(zbuf.at[pl.ds(0, _TAILN)], acc.at[pl.ds(_TAIL0, _TAILN)])

    plsc.subcore_barrier()

    @pl.loop(0, _NFULL)
    def _(c):
        off = ebase + c * _CH
        pltpu.sync_copy(edge_hbm.at[pl.ds(off, _CH)], sidx.at[0])
        pltpu.sync_copy(edge_hbm.at[pl.ds(_E + off, _CH)], didx.at[0])
        pltpu.sync_copy(g_hbm.at[sidx.at[0]], stage)
        pltpu.sync_copy(stage, acc.at[didx.at[0]], add=True)

    plsc.subcore_barrier()
    for j in range(6):
        pltpu.sync_copy(acc.at[pl.ds(row0 + j * _ZR, _ZR)], zbuf)
        pltpu.sync_copy(zbuf, out_hbm.at[cid, pl.ds(row0 + j * _ZR, _ZR)])

    @pl.when(sid == 0)
    def _():
        pltpu.sync_copy(acc.at[pl.ds(_TAIL0, _TAILN)], zbuf.at[pl.ds(0, _TAILN)])
        pltpu.sync_copy(zbuf.at[pl.ds(0, _TAILN)],
                        out_hbm.at[cid, pl.ds(_TAIL0, _TAILN)])


def _agg_partials(edge_index, g):
    k = pl.kernel(
        _agg_body,
        out_type=jax.ShapeDtypeStruct((_NC, _N, _D), _f32),
        mesh=plsc.VectorSubcoreMesh(core_axis_name="c", subcore_axis_name="s"),
        scratch_types=[
            pltpu.VMEM((1, _CH), jnp.int32),
            pltpu.VMEM((1, _CH), jnp.int32),
            pltpu.VMEM((_CH, _D), _f32),
            pltpu.VMEM((_ZR, _D), _f32),
            pltpu.VMEM_SHARED((_N, _D), _f32),
        ],
    )
    return k(edge_index, g)


_BM = 1000


def _k0_body(x_ref, w_ref, dp_ref, g_ref, dinv_ref):
    deg = dp_ref[0, :, 0:1] + dp_ref[1, :, 0:1] + 1.0
    dinv = lax.rsqrt(deg)
    h = jnp.dot(x_ref[...], w_ref[...], preferred_element_type=_f32)
    g_ref[...] = h * dinv
    dinv_ref[...] = dinv


def _k0(x, W0, degp):
    return pl.pallas_call(
        _k0_body,
        grid=(_N // _BM,),
        in_specs=[
            pl.BlockSpec((_BM, _D), lambda m: (m, 0)),
            pl.BlockSpec((_D, _D), lambda m: (0, 0)),
            pl.BlockSpec((_NC, _BM, _D), lambda m: (0, m, 0)),
        ],
        out_specs=[
            pl.BlockSpec((_BM, _D), lambda m: (m, 0)),
            pl.BlockSpec((_BM, 1), lambda m: (m, 0)),
        ],
        out_shape=[
            jax.ShapeDtypeStruct((_N, _D), _f32),
            jax.ShapeDtypeStruct((_N, 1), _f32),
        ],
    )(x, W0, degp)


def _kmid_body(p_ref, g_ref, dinv_ref, b_ref, prev_ref, w_ref,
               newprev_ref, gn_ref):
    agg = p_ref[0] + p_ref[1] + g_ref[...]
    out = dinv_ref[...] * agg + b_ref[...] + prev_ref[...]
    npv = jnp.maximum(out, 0.0)
    newprev_ref[...] = npv
    gn_ref[...] = jnp.dot(npv, w_ref[...],
                          preferred_element_type=_f32) * dinv_ref[...]


def _kmid(p, g, dinv, b, prev, Wn):
    return pl.pallas_call(
        _kmid_body,
        grid=(_N // _BM,),
        in_specs=[
            pl.BlockSpec((_NC, _BM, _D), lambda m: (0, m, 0)),
            pl.BlockSpec((_BM, _D), lambda m: (m, 0)),
            pl.BlockSpec((_BM, 1), lambda m: (m, 0)),
            pl.BlockSpec((1, _D), lambda m: (0, 0)),
            pl.BlockSpec((_BM, _D), lambda m: (m, 0)),
            pl.BlockSpec((_D, _D), lambda m: (0, 0)),
        ],
        out_specs=[
            pl.BlockSpec((_BM, _D), lambda m: (m, 0)),
            pl.BlockSpec((_BM, _D), lambda m: (m, 0)),
        ],
        out_shape=[
            jax.ShapeDtypeStruct((_N, _D), _f32),
            jax.ShapeDtypeStruct((_N, _D), _f32),
        ],
    )(p, g, dinv, b, prev, Wn)


def _k3_body(p_ref, g_ref, dinv_ref, b_ref, prev_ref, out_ref):
    agg = p_ref[0] + p_ref[1] + g_ref[...]
    out = dinv_ref[...] * agg + b_ref[...] + prev_ref[...]
    out_ref[...] = jnp.maximum(out, 0.0)


def _k3(p, g, dinv, b, prev):
    return pl.pallas_call(
        _k3_body,
        grid=(_N // _BM,),
        in_specs=[
            pl.BlockSpec((_NC, _BM, _D), lambda m: (0, m, 0)),
            pl.BlockSpec((_BM, _D), lambda m: (m, 0)),
            pl.BlockSpec((_BM, 1), lambda m: (m, 0)),
            pl.BlockSpec((1, _D), lambda m: (0, 0)),
            pl.BlockSpec((_BM, _D), lambda m: (m, 0)),
        ],
        out_specs=pl.BlockSpec((_BM, _D), lambda m: (m, 0)),
        out_shape=jax.ShapeDtypeStruct((_N, _D), _f32),
    )(p, g, dinv, b, prev)


def kernel(x, edge_index, W0, b0, W1, b1, W2, b2):
    eflat = edge_index.reshape(2 * _E)
    ones = jnp.ones((_N, _D), _f32)
    degp = _agg_partials(eflat, ones)
    g0, dinv = _k0(x, W0, degp)
    p0 = _agg_partials(eflat, g0)
    prev1, g1 = _kmid(p0, g0, dinv, b0.reshape(1, _D), x, W1)
    p1 = _agg_partials(eflat, g1)
    prev2, g2 = _kmid(p1, g1, dinv, b1.reshape(1, _D), prev1, W2)
    p2 = _agg_partials(eflat, g2)
    return _k3(p2, g2, dinv, b2.reshape(1, _D), prev2)
